# baseline (speedup 1.0000x reference)
import functools

import jax
import jax.numpy as jnp
from jax import lax
from jax.experimental import pallas as pl
from jax.experimental.pallas import tpu as pltpu
from jax.experimental.pallas import tpu_sc as plsc

_VOCAB1 = 1000001
_PAD_LEN = 1048576
_ROWS = _PAD_LEN // 128
_BLK = 1024
_NC, _NS = 2, 16
_NW = _NC * _NS
_N_DOCS = 4096
_WPD = 208
_DOCS_PER_W = _N_DOCS // _NW
_IDX_PER_W = _DOCS_PER_W * _WPD
_CHUNK = 128
_NCHUNK = _IDX_PER_W // _CHUNK


def _combine_body(s_ref, w_ref, r_ref, t_ref):
    w_adj = s_ref[0]
    r_inv = s_ref[1]
    i = pl.program_id(0)
    row = lax.broadcasted_iota(jnp.int32, (_BLK, 128), 0)
    col = lax.broadcasted_iota(jnp.int32, (_BLK, 128), 1)
    first = jnp.logical_and(i == 0, jnp.logical_and(row == 0, col == 0))
    t = (w_ref[...] + w_adj) * r_ref[...] * r_inv
    t_ref[...] = jnp.where(first, jnp.float32(0.0), t)


def _combine_tables(w2d, r2d, scal):
    return pl.pallas_call(
        _combine_body,
        grid=(_ROWS // _BLK,),
        in_specs=[
            pl.BlockSpec(memory_space=pltpu.SMEM),
            pl.BlockSpec((_BLK, 128), lambda i: (i, 0)),
            pl.BlockSpec((_BLK, 128), lambda i: (i, 0)),
        ],
        out_specs=pl.BlockSpec((_BLK, 128), lambda i: (i, 0)),
        out_shape=jax.ShapeDtypeStruct((_ROWS, 128), jnp.float32),
    )(scal, w2d, r2d)


def _gather_reduce_body(t_hbm, idx_hbm, out_hbm, idx_v, vals_v, out_v, sem):
    wid = lax.axis_index("s") * _NC + lax.axis_index("c")
    pltpu.sync_copy(idx_hbm.at[wid], idx_v)

    def fire(c, carry):
        pltpu.async_copy(t_hbm.at[idx_v.at[c]], vals_v.at[c], sem)
        return carry

    lax.fori_loop(0, _NCHUNK, fire, 0)

    def drain(c, carry):
        pltpu.make_async_copy(t_hbm.at[idx_v.at[c]], vals_v.at[c], sem).wait()
        return carry

    lax.fori_loop(0, _NCHUNK, drain, 0)

    for g in range(_DOCS_PER_W // 16):
        def rowloop(j, acc, _g=g):
            return acc + vals_v[j, pl.ds(_g * 16, 16)]

        acc0 = vals_v[0, pl.ds(g * 16, 16)]
        out_v[pl.ds(g * 16, 16)] = lax.fori_loop(1, _WPD, rowloop, acc0)
    pltpu.sync_copy(out_v, out_hbm.at[pl.ds(wid * _DOCS_PER_W, _DOCS_PER_W)])


_gather_reduce = functools.partial(
    pl.kernel,
    out_type=jax.ShapeDtypeStruct((_N_DOCS,), jnp.float32),
    mesh=plsc.VectorSubcoreMesh(
        core_axis_name="c", subcore_axis_name="s",
        num_cores=_NC, num_subcores=_NS),
    scratch_types=[
        pltpu.VMEM((_NCHUNK, _CHUNK), jnp.int32),
        pltpu.VMEM((_NCHUNK, _CHUNK), jnp.float32),
        pltpu.VMEM((_DOCS_PER_W,), jnp.float32),
        pltpu.SemaphoreType.DMA,
    ],
    compiler_params=pltpu.CompilerParams(needs_layout_passes=False),
)(_gather_reduce_body)


@jax.jit
def kernel(feat_idx, w_weight, r_weight, w_adj, r_adj):
    scal = jnp.stack([w_adj, 1.0 / r_adj]).astype(jnp.float32)
    w2d = jnp.pad(w_weight, (0, _PAD_LEN - _VOCAB1)).reshape(_ROWS, 128)
    r2d = jnp.pad(r_weight, (0, _PAD_LEN - _VOCAB1)).reshape(_ROWS, 128)
    t = _combine_tables(w2d, r2d, scal).reshape(_PAD_LEN)
    idx = jnp.pad(feat_idx, ((0, 0), (0, _WPD - feat_idx.shape[1])))
    idx3 = idx.reshape(_NW, _DOCS_PER_W, _WPD).transpose(0, 2, 1)
    return _gather_reduce(t, idx3)


# speedup vs baseline: 3.4167x; 3.4167x over previous
import functools

import jax
import jax.numpy as jnp
from jax import lax
from jax.experimental import pallas as pl
from jax.experimental.pallas import tpu as pltpu
from jax.experimental.pallas import tpu_sc as plsc

_VOCAB1 = 1000001
_PAD_LEN = 1048576
_ROWS = _PAD_LEN // 128
_BLK = 1024
_NC, _NS = 2, 16
_NW = _NC * _NS
_N_DOCS = 4096
_WPD = 208
_DOCS_PER_W = _N_DOCS // _NW
_IDX_PER_W = _DOCS_PER_W * _WPD
_CHUNK = 128
_NCHUNK = _IDX_PER_W // _CHUNK


def _combine_body(s_ref, w_ref, r_ref, t_ref):
    w_adj = s_ref[0]
    r_inv = s_ref[1]
    i = pl.program_id(0)
    row = lax.broadcasted_iota(jnp.int32, (_BLK, 128), 0)
    col = lax.broadcasted_iota(jnp.int32, (_BLK, 128), 1)
    first = jnp.logical_and(i == 0, jnp.logical_and(row == 0, col == 0))
    t = (w_ref[...] + w_adj) * r_ref[...] * r_inv
    t_ref[...] = jnp.where(first, jnp.float32(0.0), t)


def _combine_tables(w2d, r2d, scal):
    return pl.pallas_call(
        _combine_body,
        grid=(_ROWS // _BLK,),
        in_specs=[
            pl.BlockSpec(memory_space=pltpu.SMEM),
            pl.BlockSpec((_BLK, 128), lambda i: (i, 0)),
            pl.BlockSpec((_BLK, 128), lambda i: (i, 0)),
        ],
        out_specs=pl.BlockSpec((_BLK, 128), lambda i: (i, 0)),
        out_shape=jax.ShapeDtypeStruct((_ROWS, 128), jnp.float32),
    )(scal, w2d, r2d)


def _gather_reduce_body(t_hbm, idx_hbm, out_hbm, t_sh, idx_v, vals_v, out_v,
                        sem):
    sid = lax.axis_index("s")
    wid = sid * _NC + lax.axis_index("c")
    seg = _PAD_LEN // _NS
    pltpu.sync_copy(t_hbm.at[pl.ds(sid * seg, seg)],
                    t_sh.at[pl.ds(sid * seg, seg)])
    pltpu.sync_copy(idx_hbm.at[wid], idx_v)
    plsc.subcore_barrier()

    def fire(c, carry):
        pltpu.async_copy(t_sh.at[idx_v.at[c]], vals_v.at[c], sem)
        return carry

    lax.fori_loop(0, _NCHUNK, fire, 0)

    def drain(c, carry):
        pltpu.make_async_copy(t_sh.at[idx_v.at[c]], vals_v.at[c], sem).wait()
        return carry

    lax.fori_loop(0, _NCHUNK, drain, 0)

    for g in range(_DOCS_PER_W // 16):
        def rowloop(j, acc, _g=g):
            return acc + vals_v[j, pl.ds(_g * 16, 16)]

        acc0 = vals_v[0, pl.ds(g * 16, 16)]
        out_v[pl.ds(g * 16, 16)] = lax.fori_loop(1, _WPD, rowloop, acc0)
    pltpu.sync_copy(out_v, out_hbm.at[pl.ds(wid * _DOCS_PER_W, _DOCS_PER_W)])


_gather_reduce = functools.partial(
    pl.kernel,
    out_type=jax.ShapeDtypeStruct((_N_DOCS,), jnp.float32),
    mesh=plsc.VectorSubcoreMesh(
        core_axis_name="c", subcore_axis_name="s",
        num_cores=_NC, num_subcores=_NS),
    scratch_types=[
        pltpu.VMEM_SHARED((_PAD_LEN,), jnp.float32),
        pltpu.VMEM((_NCHUNK, _CHUNK), jnp.int32),
        pltpu.VMEM((_NCHUNK, _CHUNK), jnp.float32),
        pltpu.VMEM((_DOCS_PER_W,), jnp.float32),
        pltpu.SemaphoreType.DMA,
    ],
    compiler_params=pltpu.CompilerParams(needs_layout_passes=False),
)(_gather_reduce_body)


@jax.jit
def kernel(feat_idx, w_weight, r_weight, w_adj, r_adj):
    scal = jnp.stack([w_adj, 1.0 / r_adj]).astype(jnp.float32)
    w2d = jnp.pad(w_weight, (0, _PAD_LEN - _VOCAB1)).reshape(_ROWS, 128)
    r2d = jnp.pad(r_weight, (0, _PAD_LEN - _VOCAB1)).reshape(_ROWS, 128)
    t = _combine_tables(w2d, r2d, scal).reshape(_PAD_LEN)
    idx = jnp.pad(feat_idx, ((0, 0), (0, _WPD - feat_idx.shape[1])))
    idx3 = idx.reshape(_NW, _DOCS_PER_W, _WPD).transpose(0, 2, 1)
    return _gather_reduce(t, idx3)


# speedup vs baseline: 3.7078x; 1.0852x over previous
import functools

import jax
import jax.numpy as jnp
from jax import lax
from jax.experimental import pallas as pl
from jax.experimental.pallas import tpu as pltpu
from jax.experimental.pallas import tpu_sc as plsc

_VOCAB1 = 1000001
_PAD_LEN = 1048576
_ROWS = _PAD_LEN // 128
_BLK = 1024
_NC, _NS = 2, 16
_NW = _NC * _NS
_N_DOCS = 4096
_WPD = 200
_DOCS_PER_W = _N_DOCS // _NW
_IDX_PER_W = _DOCS_PER_W * _WPD
_CHUNK = 128
_NCHUNK = _IDX_PER_W // _CHUNK


def _combine_body(s_ref, w_ref, r_ref, t_ref):
    w_adj = s_ref[0]
    r_inv = s_ref[1]
    i = pl.program_id(0)
    row = lax.broadcasted_iota(jnp.int32, (_BLK, 128), 0)
    col = lax.broadcasted_iota(jnp.int32, (_BLK, 128), 1)
    first = jnp.logical_and(i == 0, jnp.logical_and(row == 0, col == 0))
    t = (w_ref[...] + w_adj) * r_ref[...] * r_inv
    t_ref[...] = jnp.where(first, jnp.float32(0.0), t)


def _combine_tables(w2d, r2d, scal):
    return pl.pallas_call(
        _combine_body,
        grid=(_ROWS // _BLK,),
        in_specs=[
            pl.BlockSpec(memory_space=pltpu.SMEM),
            pl.BlockSpec((_BLK, 128), lambda i: (i, 0)),
            pl.BlockSpec((_BLK, 128), lambda i: (i, 0)),
        ],
        out_specs=pl.BlockSpec((_BLK, 128), lambda i: (i, 0)),
        out_shape=jax.ShapeDtypeStruct((_ROWS, 128), jnp.float32),
    )(scal, w2d, r2d)


def _gather_reduce_body(t_hbm, idx_hbm, out_hbm, t_sh, idx_v, vals_v, tmp_v,
                        out_v, sem):
    sid = lax.axis_index("s")
    wid = sid * _NC + lax.axis_index("c")
    seg = _PAD_LEN // _NS
    pltpu.sync_copy(t_hbm.at[pl.ds(sid * seg, seg)],
                    t_sh.at[pl.ds(sid * seg, seg)])
    pltpu.sync_copy(idx_hbm.at[wid], idx_v)
    plsc.subcore_barrier()

    def fire(c, carry):
        pltpu.async_copy(t_sh.at[idx_v.at[c]],
                         vals_v.at[pl.ds(c * _CHUNK, _CHUNK)], sem)
        return carry

    lax.fori_loop(0, _NCHUNK, fire, 0)

    def drain(c, carry):
        pltpu.make_async_copy(t_sh.at[idx_v.at[c]],
                              vals_v.at[pl.ds(c * _CHUNK, _CHUNK)], sem).wait()
        return carry

    lax.fori_loop(0, _NCHUNK, drain, 0)

    lane = lax.iota(jnp.int32, 16)
    tail_mask = lane < 8
    lanes16 = lane * 16

    def group(g, carry):
        for l in range(16):
            base = (g * 16 + l) * _WPD
            acc = vals_v[pl.ds(base, 16)]
            for j in range(1, 12):
                acc = acc + vals_v[pl.ds(base + j * 16, 16)]
            tail = vals_v[pl.ds(base + 192, 16)]
            acc = acc + jnp.where(tail_mask, tail, jnp.float32(0.0))
            tmp_v[pl.ds(l * 16, 16)] = acc
        tot = plsc.load_gather(tmp_v, [lanes16])
        for k in range(1, 16):
            tot = tot + plsc.load_gather(tmp_v, [lanes16 + k])
        out_v[pl.ds(g * 16, 16)] = tot
        return carry

    lax.fori_loop(0, _DOCS_PER_W // 16, group, 0)
    pltpu.sync_copy(out_v, out_hbm.at[pl.ds(wid * _DOCS_PER_W, _DOCS_PER_W)])


_gather_reduce = functools.partial(
    pl.kernel,
    out_type=jax.ShapeDtypeStruct((_N_DOCS,), jnp.float32),
    mesh=plsc.VectorSubcoreMesh(
        core_axis_name="c", subcore_axis_name="s",
        num_cores=_NC, num_subcores=_NS),
    scratch_types=[
        pltpu.VMEM_SHARED((_PAD_LEN,), jnp.float32),
        pltpu.VMEM((_NCHUNK, _CHUNK), jnp.int32),
        pltpu.VMEM((_IDX_PER_W + 16,), jnp.float32),
        pltpu.VMEM((256,), jnp.float32),
        pltpu.VMEM((_DOCS_PER_W,), jnp.float32),
        pltpu.SemaphoreType.DMA,
    ],
    compiler_params=pltpu.CompilerParams(needs_layout_passes=False),
)(_gather_reduce_body)


@jax.jit
def kernel(feat_idx, w_weight, r_weight, w_adj, r_adj):
    scal = jnp.stack([w_adj, 1.0 / r_adj]).astype(jnp.float32)
    w2d = jnp.pad(w_weight, (0, _PAD_LEN - _VOCAB1)).reshape(_ROWS, 128)
    r2d = jnp.pad(r_weight, (0, _PAD_LEN - _VOCAB1)).reshape(_ROWS, 128)
    t = _combine_tables(w2d, r2d, scal).reshape(_PAD_LEN)
    idx3 = feat_idx.reshape(_NW, _NCHUNK, _CHUNK)
    return _gather_reduce(t, idx3)
